# baseline (speedup 1.0000x reference)
import functools

import jax
import jax.numpy as jnp
import numpy as np
from jax import lax
from jax.experimental import pallas as pl
from jax.experimental.pallas import tpu as pltpu
from jax.experimental.pallas import tpu_sc as plsc

_D_IN = 768
_D = 1024
_L = 16
_CH = 256
_ROWS_PER_BLOCK = 512

_NW = 32
_SC_CHUNK = 32768


def _hadamard_f32(n: int) -> np.ndarray:
    h = np.array([[1.0]], dtype=np.float32)
    while h.shape[0] < n:
        h = np.block([[h, h], [h, -h]])
    return h


def _fwd_body(x_ref, flips_ref, h_ref, o_ref):
    hs = h_ref[...]
    z = x_ref[...] * flips_ref[...]
    dims = (((1,), (0,)), ((), ()))

    def hmul(za):
        return lax.dot_general(za, hs, dims,
                               precision=lax.Precision.HIGHEST,
                               preferred_element_type=jnp.float32)

    m0 = hmul(z[:, 0 * _CH:1 * _CH])
    m1 = hmul(z[:, 1 * _CH:2 * _CH])
    m2 = hmul(z[:, 2 * _CH:3 * _CH])
    b0 = m0 + m1
    b1 = m0 - m1
    o_ref[:, 0 * _CH:1 * _CH] = b0 + m2
    o_ref[:, 1 * _CH:2 * _CH] = b1 + m2
    o_ref[:, 2 * _CH:3 * _CH] = b0 - m2
    o_ref[:, 3 * _CH:4 * _CH] = b1 - m2


def _inv_body(q_ref, flips_ref, h_ref, o_ref):
    hs = h_ref[...]
    q = q_ref[...]
    q0 = q[:, 0 * _CH:1 * _CH]
    q1 = q[:, 1 * _CH:2 * _CH]
    q2 = q[:, 2 * _CH:3 * _CH]
    q3 = q[:, 3 * _CH:4 * _CH]
    u0 = q0 + q1
    u1 = q0 - q1
    u2 = q2 + q3
    u3 = q2 - q3
    dims = (((1,), (0,)), ((), ()))

    def hmul(pa):
        return lax.dot_general(pa, hs, dims,
                               preferred_element_type=jnp.float32)

    f = flips_ref[...]
    o_ref[:, 0 * _CH:1 * _CH] = hmul(u0 + u2) * f[:, 0 * _CH:1 * _CH]
    o_ref[:, 1 * _CH:2 * _CH] = hmul(u1 + u3) * f[:, 1 * _CH:2 * _CH]
    o_ref[:, 2 * _CH:3 * _CH] = hmul(u0 - u2) * f[:, 2 * _CH:3 * _CH]


_SC_ROWS = 32


def _sc_quant_body(y_hbm, bp_hbm, cent_hbm, out_hbm,
                   bpv, centv, b0, b1, b2,
                   si0, si1, si2, so0, so1, so2):
    n_rows = y_hbm.shape[0]
    per_w = n_rows // _NW
    n_chunks = per_w // _SC_ROWS
    wid = lax.axis_index("s") * 2 + lax.axis_index("c")
    base = wid * per_w

    pltpu.sync_copy(bp_hbm, bpv)
    pltpu.sync_copy(cent_hbm, centv)

    bufs = (b0, b1, b2)
    sin = (si0, si1, si2)
    sout = (so0, so1, so2)

    def splat(ref, i):
        return plsc.load_gather(ref, [jnp.full((16,), i, jnp.int32)])

    bp1v = splat(bpv, 1)
    bp3v = splat(bpv, 3)
    bp5v = splat(bpv, 5)
    bp7v = splat(bpv, 7)
    bp9v = splat(bpv, 9)
    bp11v = splat(bpv, 11)
    bp13v = splat(bpv, 13)
    c8 = jnp.full((16,), 8, jnp.int32)
    c0 = jnp.full((16,), 0, jnp.int32)
    c4 = jnp.full((16,), 4, jnp.int32)
    c2 = jnp.full((16,), 2, jnp.int32)
    c1 = jnp.full((16,), 1, jnp.int32)

    def quantize_chunk(buf):
        @plsc.parallel_loop(0, _SC_ROWS * (_D // 16), 1, unroll=8)
        def _(i):
            row = lax.shift_right_logical(i, 6)
            col = lax.shift_left(jnp.bitwise_and(i, 63), 4)
            yv = buf[row, pl.ds(col, 16)]
            m8 = yv > bp7v
            lo = jnp.where(m8, c8, c0)
            m4 = yv > jnp.where(m8, bp11v, bp3v)
            lo = lo + jnp.where(m4, c4, c0)
            probe2 = jnp.where(m8, jnp.where(m4, bp13v, bp9v),
                               jnp.where(m4, bp5v, bp1v))
            lo = lo + jnp.where(yv > probe2, c2, c0)
            lo = lo + jnp.where(yv > plsc.load_gather(bpv, [lo]), c1, c0)
            buf[row, pl.ds(col, 16)] = plsc.load_gather(centv, [lo])

    in_h = [None, None, None]
    out_h = [None, None, None]
    in_h[0] = pltpu.async_copy(y_hbm.at[pl.ds(base, _SC_ROWS)], b0, si0)
    in_h[1] = pltpu.async_copy(
        y_hbm.at[pl.ds(base + _SC_ROWS, _SC_ROWS)], b1, si1)
    for c in range(n_chunks):
        k = c % 3
        in_h[k].wait()
        quantize_chunk(bufs[k])
        out_h[k] = pltpu.async_copy(
            bufs[k], out_hbm.at[pl.ds(base + c * _SC_ROWS, _SC_ROWS)],
            sout[k])
        if c + 2 < n_chunks:
            k2 = (c + 2) % 3
            if out_h[k2] is not None:
                out_h[k2].wait()
            in_h[k2] = pltpu.async_copy(
                y_hbm.at[pl.ds(base + (c + 2) * _SC_ROWS, _SC_ROWS)],
                bufs[k2], sin[k2])
    for k in range(3):
        if out_h[k] is not None:
            out_h[k].wait()


def _run_slice(xf, flips768, hs, bp128, cent128):
    n = xf.shape[0]
    r = _ROWS_PER_BLOCK
    grid = (n // r,)

    y = pl.pallas_call(
        _fwd_body,
        grid=grid,
        in_specs=[
            pl.BlockSpec((r, _D_IN), lambda i: (i, 0)),
            pl.BlockSpec((1, _D_IN), lambda i: (0, 0)),
            pl.BlockSpec((_CH, _CH), lambda i: (0, 0)),
        ],
        out_specs=pl.BlockSpec((r, _D), lambda i: (i, 0)),
        out_shape=jax.ShapeDtypeStruct((n, _D), jnp.float32),
        compiler_params=pltpu.CompilerParams(
            dimension_semantics=("arbitrary",)),
    )(xf, flips768, hs)

    mesh = plsc.VectorSubcoreMesh(core_axis_name="c", subcore_axis_name="s")
    sc_quant = functools.partial(
        pl.kernel,
        mesh=mesh,
        compiler_params=pltpu.CompilerParams(
            needs_layout_passes=False, use_tc_tiling_on_sc=True),
        out_type=jax.ShapeDtypeStruct((n, _D), jnp.float32),
        scratch_types=[
            pltpu.VMEM((128,), jnp.float32),
            pltpu.VMEM((128,), jnp.float32),
            pltpu.VMEM((_SC_ROWS, _D), jnp.float32),
            pltpu.VMEM((_SC_ROWS, _D), jnp.float32),
            pltpu.VMEM((_SC_ROWS, _D), jnp.float32),
            pltpu.SemaphoreType.DMA,
            pltpu.SemaphoreType.DMA,
            pltpu.SemaphoreType.DMA,
            pltpu.SemaphoreType.DMA,
            pltpu.SemaphoreType.DMA,
            pltpu.SemaphoreType.DMA,
        ],
    )(_sc_quant_body)
    q = sc_quant(y, bp128, cent128)

    out = pl.pallas_call(
        _inv_body,
        grid=grid,
        in_specs=[
            pl.BlockSpec((r, _D), lambda i: (i, 0)),
            pl.BlockSpec((1, _D_IN), lambda i: (0, 0)),
            pl.BlockSpec((_CH, _CH), lambda i: (0, 0)),
        ],
        out_specs=pl.BlockSpec((r, _D_IN), lambda i: (i, 0)),
        out_shape=jax.ShapeDtypeStruct((n, _D_IN), jnp.float32),
        compiler_params=pltpu.CompilerParams(
            dimension_semantics=("arbitrary",)),
    )(q, flips768, hs)
    return out


@jax.jit
def kernel(x, flips, bp, cent):
    orig_dtype = x.dtype
    n = x.shape[0] * x.shape[1]
    xf = x.reshape(n, _D_IN).astype(jnp.float32)
    hs = jnp.asarray(_hadamard_f32(_CH) * (1.0 / 32.0))
    flips768 = flips[:_D_IN].reshape(1, _D_IN)
    bp128 = jnp.concatenate([bp, jnp.full((113,), jnp.inf, jnp.float32)])
    cent128 = jnp.concatenate([cent, jnp.zeros((112,), jnp.float32)])

    ns = 1
    step = n // ns
    outs = [_run_slice(xf[i * step:(i + 1) * step], flips768, hs, bp128,
                       cent128) for i in range(ns)]
    out = jnp.concatenate(outs, axis=0)
    return out.reshape(x.shape).astype(orig_dtype)


# speedup vs baseline: 1.0328x; 1.0328x over previous
import functools

import jax
import jax.numpy as jnp
import numpy as np
from jax import lax
from jax.experimental import pallas as pl
from jax.experimental.pallas import tpu as pltpu
from jax.experimental.pallas import tpu_sc as plsc

_D_IN = 768
_D = 1024
_L = 16
_CH = 256
_ROWS_PER_BLOCK = 512

_NW = 32
_SC_CHUNK = 32768


def _hadamard_f32(n: int) -> np.ndarray:
    h = np.array([[1.0]], dtype=np.float32)
    while h.shape[0] < n:
        h = np.block([[h, h], [h, -h]])
    return h


def _fwd_body(x_ref, flips_ref, h_ref, o_ref):
    hs = h_ref[...]
    z = x_ref[...] * flips_ref[...]
    dims = (((1,), (0,)), ((), ()))

    def hmul(za):
        return lax.dot_general(za, hs, dims,
                               precision=lax.Precision.HIGHEST,
                               preferred_element_type=jnp.float32)

    m0 = hmul(z[:, 0 * _CH:1 * _CH])
    m1 = hmul(z[:, 1 * _CH:2 * _CH])
    m2 = hmul(z[:, 2 * _CH:3 * _CH])
    b0 = m0 + m1
    b1 = m0 - m1
    o_ref[:, 0 * _CH:1 * _CH] = b0 + m2
    o_ref[:, 1 * _CH:2 * _CH] = b1 + m2
    o_ref[:, 2 * _CH:3 * _CH] = b0 - m2
    o_ref[:, 3 * _CH:4 * _CH] = b1 - m2


def _inv_body(q_ref, flips_ref, h_ref, o_ref):
    hs = h_ref[...]
    q = q_ref[...]
    q0 = q[:, 0 * _CH:1 * _CH]
    q1 = q[:, 1 * _CH:2 * _CH]
    q2 = q[:, 2 * _CH:3 * _CH]
    q3 = q[:, 3 * _CH:4 * _CH]
    u0 = q0 + q1
    u1 = q0 - q1
    u2 = q2 + q3
    u3 = q2 - q3
    dims = (((1,), (0,)), ((), ()))

    def hmul(pa):
        return lax.dot_general(pa, hs, dims,
                               preferred_element_type=jnp.float32)

    f = flips_ref[...]
    o_ref[:, 0 * _CH:1 * _CH] = hmul(u0 + u2) * f[:, 0 * _CH:1 * _CH]
    o_ref[:, 1 * _CH:2 * _CH] = hmul(u1 + u3) * f[:, 1 * _CH:2 * _CH]
    o_ref[:, 2 * _CH:3 * _CH] = hmul(u0 - u2) * f[:, 2 * _CH:3 * _CH]


_SC_ROWS = 32


def _sc_quant_body(y_hbm, bp_hbm, cent_hbm, out_hbm,
                   bpv, centv, b0, b1, b2,
                   si0, si1, si2, so0, so1, so2):
    n_rows = y_hbm.shape[0]
    per_w = n_rows // _NW
    n_chunks = per_w // _SC_ROWS
    wid = lax.axis_index("s") * 2 + lax.axis_index("c")
    base = wid * per_w

    pltpu.sync_copy(bp_hbm, bpv)
    pltpu.sync_copy(cent_hbm, centv)

    bufs = (b0, b1, b2)
    sin = (si0, si1, si2)
    sout = (so0, so1, so2)

    def splat(ref, i):
        return plsc.load_gather(ref, [jnp.full((16,), i, jnp.int32)])

    bp1v = splat(bpv, 1)
    bp3v = splat(bpv, 3)
    bp5v = splat(bpv, 5)
    bp7v = splat(bpv, 7)
    bp9v = splat(bpv, 9)
    bp11v = splat(bpv, 11)
    bp13v = splat(bpv, 13)
    c8 = jnp.full((16,), 8, jnp.int32)
    c0 = jnp.full((16,), 0, jnp.int32)
    c4 = jnp.full((16,), 4, jnp.int32)
    c2 = jnp.full((16,), 2, jnp.int32)
    c1 = jnp.full((16,), 1, jnp.int32)

    def quantize_chunk(buf):
        @plsc.parallel_loop(0, _SC_ROWS * (_D // 16), 1, unroll=8)
        def _(i):
            row = lax.shift_right_logical(i, 6)
            col = lax.shift_left(jnp.bitwise_and(i, 63), 4)
            yv = buf[row, pl.ds(col, 16)]
            m8 = yv > bp7v
            lo = jnp.where(m8, c8, c0)
            probe = jnp.where(m8, bp11v, bp3v)
            lo = lo + jnp.where(yv > probe, c4, c0)
            lo = lo + jnp.where(yv > plsc.load_gather(bpv, [lo + 1]), c2, c0)
            lo = lo + jnp.where(yv > plsc.load_gather(bpv, [lo]), c1, c0)
            buf[row, pl.ds(col, 16)] = plsc.load_gather(centv, [lo])

    in_h = [None, None, None]
    out_h = [None, None, None]
    in_h[0] = pltpu.async_copy(y_hbm.at[pl.ds(base, _SC_ROWS)], b0, si0)
    in_h[1] = pltpu.async_copy(
        y_hbm.at[pl.ds(base + _SC_ROWS, _SC_ROWS)], b1, si1)
    for c in range(n_chunks):
        k = c % 3
        in_h[k].wait()
        quantize_chunk(bufs[k])
        out_h[k] = pltpu.async_copy(
            bufs[k], out_hbm.at[pl.ds(base + c * _SC_ROWS, _SC_ROWS)],
            sout[k])
        if c + 2 < n_chunks:
            k2 = (c + 2) % 3
            if out_h[k2] is not None:
                out_h[k2].wait()
            in_h[k2] = pltpu.async_copy(
                y_hbm.at[pl.ds(base + (c + 2) * _SC_ROWS, _SC_ROWS)],
                bufs[k2], sin[k2])
    for k in range(3):
        if out_h[k] is not None:
            out_h[k].wait()


def _run_slice(xf, flips768, hs, bp128, cent128):
    n = xf.shape[0]
    r = _ROWS_PER_BLOCK
    grid = (n // r,)

    y = pl.pallas_call(
        _fwd_body,
        grid=grid,
        in_specs=[
            pl.BlockSpec((r, _D_IN), lambda i: (i, 0)),
            pl.BlockSpec((1, _D_IN), lambda i: (0, 0)),
            pl.BlockSpec((_CH, _CH), lambda i: (0, 0)),
        ],
        out_specs=pl.BlockSpec((r, _D), lambda i: (i, 0)),
        out_shape=jax.ShapeDtypeStruct((n, _D), jnp.float32),
        compiler_params=pltpu.CompilerParams(
            dimension_semantics=("arbitrary",)),
    )(xf, flips768, hs)

    mesh = plsc.VectorSubcoreMesh(core_axis_name="c", subcore_axis_name="s")
    sc_quant = functools.partial(
        pl.kernel,
        mesh=mesh,
        compiler_params=pltpu.CompilerParams(
            needs_layout_passes=False, use_tc_tiling_on_sc=True),
        out_type=jax.ShapeDtypeStruct((n, _D), jnp.float32),
        scratch_types=[
            pltpu.VMEM((128,), jnp.float32),
            pltpu.VMEM((128,), jnp.float32),
            pltpu.VMEM((_SC_ROWS, _D), jnp.float32),
            pltpu.VMEM((_SC_ROWS, _D), jnp.float32),
            pltpu.VMEM((_SC_ROWS, _D), jnp.float32),
            pltpu.SemaphoreType.DMA,
            pltpu.SemaphoreType.DMA,
            pltpu.SemaphoreType.DMA,
            pltpu.SemaphoreType.DMA,
            pltpu.SemaphoreType.DMA,
            pltpu.SemaphoreType.DMA,
        ],
    )(_sc_quant_body)
    q = sc_quant(y, bp128, cent128)

    out = pl.pallas_call(
        _inv_body,
        grid=grid,
        in_specs=[
            pl.BlockSpec((r, _D), lambda i: (i, 0)),
            pl.BlockSpec((1, _D_IN), lambda i: (0, 0)),
            pl.BlockSpec((_CH, _CH), lambda i: (0, 0)),
        ],
        out_specs=pl.BlockSpec((r, _D_IN), lambda i: (i, 0)),
        out_shape=jax.ShapeDtypeStruct((n, _D_IN), jnp.float32),
        compiler_params=pltpu.CompilerParams(
            dimension_semantics=("arbitrary",)),
    )(q, flips768, hs)
    return out


@jax.jit
def kernel(x, flips, bp, cent):
    orig_dtype = x.dtype
    n = x.shape[0] * x.shape[1]
    xf = x.reshape(n, _D_IN).astype(jnp.float32)
    hs = jnp.asarray(_hadamard_f32(_CH) * (1.0 / 32.0))
    flips768 = flips[:_D_IN].reshape(1, _D_IN)
    bp128 = jnp.concatenate([bp, jnp.full((113,), jnp.inf, jnp.float32)])
    cent128 = jnp.concatenate([cent, jnp.zeros((112,), jnp.float32)])

    ns = 1
    step = n // ns
    outs = [_run_slice(xf[i * step:(i + 1) * step], flips768, hs, bp128,
                       cent128) for i in range(ns)]
    out = jnp.concatenate(outs, axis=0)
    return out.reshape(x.shape).astype(orig_dtype)


# speedup vs baseline: 1.0918x; 1.0572x over previous
import functools

import jax
import jax.numpy as jnp
import numpy as np
from jax import lax
from jax.experimental import pallas as pl
from jax.experimental.pallas import tpu as pltpu
from jax.experimental.pallas import tpu_sc as plsc

_D_IN = 768
_D = 1024
_L = 16
_CH = 256
_ROWS_PER_BLOCK = 512

_NW = 32
_SC_CHUNK = 32768


def _hadamard_f32(n: int) -> np.ndarray:
    h = np.array([[1.0]], dtype=np.float32)
    while h.shape[0] < n:
        h = np.block([[h, h], [h, -h]])
    return h


def _fwd_body(x_ref, flips_ref, h_ref, o_ref):
    hs = h_ref[...]
    z = x_ref[...] * flips_ref[...]
    dims = (((1,), (0,)), ((), ()))

    hb = hs.astype(jnp.bfloat16)

    def hmul(za):
        za_hi = za.astype(jnp.bfloat16)
        za_lo = (za - za_hi.astype(jnp.float32)).astype(jnp.bfloat16)
        return (lax.dot_general(za_hi, hb, dims,
                                preferred_element_type=jnp.float32)
                + lax.dot_general(za_lo, hb, dims,
                                  preferred_element_type=jnp.float32))

    m0 = hmul(z[:, 0 * _CH:1 * _CH])
    m1 = hmul(z[:, 1 * _CH:2 * _CH])
    m2 = hmul(z[:, 2 * _CH:3 * _CH])
    b0 = m0 + m1
    b1 = m0 - m1
    o_ref[:, 0 * _CH:1 * _CH] = b0 + m2
    o_ref[:, 1 * _CH:2 * _CH] = b1 + m2
    o_ref[:, 2 * _CH:3 * _CH] = b0 - m2
    o_ref[:, 3 * _CH:4 * _CH] = b1 - m2


def _inv_body(q_ref, flips_ref, h_ref, o_ref):
    hs = h_ref[...]
    q = q_ref[...]
    q0 = q[:, 0 * _CH:1 * _CH]
    q1 = q[:, 1 * _CH:2 * _CH]
    q2 = q[:, 2 * _CH:3 * _CH]
    q3 = q[:, 3 * _CH:4 * _CH]
    u0 = q0 + q1
    u1 = q0 - q1
    u2 = q2 + q3
    u3 = q2 - q3
    dims = (((1,), (0,)), ((), ()))

    def hmul(pa):
        return lax.dot_general(pa, hs, dims,
                               preferred_element_type=jnp.float32)

    f = flips_ref[...]
    o_ref[:, 0 * _CH:1 * _CH] = hmul(u0 + u2) * f[:, 0 * _CH:1 * _CH]
    o_ref[:, 1 * _CH:2 * _CH] = hmul(u1 + u3) * f[:, 1 * _CH:2 * _CH]
    o_ref[:, 2 * _CH:3 * _CH] = hmul(u0 - u2) * f[:, 2 * _CH:3 * _CH]


_SC_ROWS = 32


def _sc_quant_body(y_hbm, bp_hbm, cent_hbm, out_hbm,
                   bpv, centv, b0, b1, b2,
                   si0, si1, si2, so0, so1, so2):
    n_rows = y_hbm.shape[0]
    per_w = n_rows // _NW
    n_chunks = per_w // _SC_ROWS
    wid = lax.axis_index("s") * 2 + lax.axis_index("c")
    base = wid * per_w

    pltpu.sync_copy(bp_hbm, bpv)
    pltpu.sync_copy(cent_hbm, centv)

    bufs = (b0, b1, b2)
    sin = (si0, si1, si2)
    sout = (so0, so1, so2)

    def splat(ref, i):
        return plsc.load_gather(ref, [jnp.full((16,), i, jnp.int32)])

    bp1v = splat(bpv, 1)
    bp3v = splat(bpv, 3)
    bp5v = splat(bpv, 5)
    bp7v = splat(bpv, 7)
    bp9v = splat(bpv, 9)
    bp11v = splat(bpv, 11)
    bp13v = splat(bpv, 13)
    c8 = jnp.full((16,), 8, jnp.int32)
    c0 = jnp.full((16,), 0, jnp.int32)
    c4 = jnp.full((16,), 4, jnp.int32)
    c2 = jnp.full((16,), 2, jnp.int32)
    c1 = jnp.full((16,), 1, jnp.int32)

    def quantize_chunk(buf):
        @plsc.parallel_loop(0, _SC_ROWS * (_D // 16), 1, unroll=8)
        def _(i):
            row = lax.shift_right_logical(i, 6)
            col = lax.shift_left(jnp.bitwise_and(i, 63), 4)
            yv = buf[row, pl.ds(col, 16)]
            m8 = yv > bp7v
            lo = jnp.where(m8, c8, c0)
            probe = jnp.where(m8, bp11v, bp3v)
            lo = lo + jnp.where(yv > probe, c4, c0)
            lo = lo + jnp.where(yv > plsc.load_gather(bpv, [lo + 1]), c2, c0)
            lo = lo + jnp.where(yv > plsc.load_gather(bpv, [lo]), c1, c0)
            buf[row, pl.ds(col, 16)] = plsc.load_gather(centv, [lo])

    in_h = [None, None, None]
    out_h = [None, None, None]
    in_h[0] = pltpu.async_copy(y_hbm.at[pl.ds(base, _SC_ROWS)], b0, si0)
    in_h[1] = pltpu.async_copy(
        y_hbm.at[pl.ds(base + _SC_ROWS, _SC_ROWS)], b1, si1)
    for c in range(n_chunks):
        k = c % 3
        in_h[k].wait()
        quantize_chunk(bufs[k])
        out_h[k] = pltpu.async_copy(
            bufs[k], out_hbm.at[pl.ds(base + c * _SC_ROWS, _SC_ROWS)],
            sout[k])
        if c + 2 < n_chunks:
            k2 = (c + 2) % 3
            if out_h[k2] is not None:
                out_h[k2].wait()
            in_h[k2] = pltpu.async_copy(
                y_hbm.at[pl.ds(base + (c + 2) * _SC_ROWS, _SC_ROWS)],
                bufs[k2], sin[k2])
    for k in range(3):
        if out_h[k] is not None:
            out_h[k].wait()


def _run_slice(xf, flips768, hs, bp128, cent128):
    n = xf.shape[0]
    r = _ROWS_PER_BLOCK
    grid = (n // r,)

    y = pl.pallas_call(
        _fwd_body,
        grid=grid,
        in_specs=[
            pl.BlockSpec((r, _D_IN), lambda i: (i, 0)),
            pl.BlockSpec((1, _D_IN), lambda i: (0, 0)),
            pl.BlockSpec((_CH, _CH), lambda i: (0, 0)),
        ],
        out_specs=pl.BlockSpec((r, _D), lambda i: (i, 0)),
        out_shape=jax.ShapeDtypeStruct((n, _D), jnp.float32),
        compiler_params=pltpu.CompilerParams(
            dimension_semantics=("arbitrary",)),
    )(xf, flips768, hs)

    mesh = plsc.VectorSubcoreMesh(core_axis_name="c", subcore_axis_name="s")
    sc_quant = functools.partial(
        pl.kernel,
        mesh=mesh,
        compiler_params=pltpu.CompilerParams(
            needs_layout_passes=False, use_tc_tiling_on_sc=True),
        out_type=jax.ShapeDtypeStruct((n, _D), jnp.float32),
        scratch_types=[
            pltpu.VMEM((128,), jnp.float32),
            pltpu.VMEM((128,), jnp.float32),
            pltpu.VMEM((_SC_ROWS, _D), jnp.float32),
            pltpu.VMEM((_SC_ROWS, _D), jnp.float32),
            pltpu.VMEM((_SC_ROWS, _D), jnp.float32),
            pltpu.SemaphoreType.DMA,
            pltpu.SemaphoreType.DMA,
            pltpu.SemaphoreType.DMA,
            pltpu.SemaphoreType.DMA,
            pltpu.SemaphoreType.DMA,
            pltpu.SemaphoreType.DMA,
        ],
    )(_sc_quant_body)
    q = sc_quant(y, bp128, cent128)

    out = pl.pallas_call(
        _inv_body,
        grid=grid,
        in_specs=[
            pl.BlockSpec((r, _D), lambda i: (i, 0)),
            pl.BlockSpec((1, _D_IN), lambda i: (0, 0)),
            pl.BlockSpec((_CH, _CH), lambda i: (0, 0)),
        ],
        out_specs=pl.BlockSpec((r, _D_IN), lambda i: (i, 0)),
        out_shape=jax.ShapeDtypeStruct((n, _D_IN), jnp.float32),
        compiler_params=pltpu.CompilerParams(
            dimension_semantics=("arbitrary",)),
    )(q, flips768, hs)
    return out


@jax.jit
def kernel(x, flips, bp, cent):
    orig_dtype = x.dtype
    n = x.shape[0] * x.shape[1]
    xf = x.reshape(n, _D_IN).astype(jnp.float32)
    hs = jnp.asarray(_hadamard_f32(_CH) * (1.0 / 32.0))
    flips768 = flips[:_D_IN].reshape(1, _D_IN)
    bp128 = jnp.concatenate([bp, jnp.full((113,), jnp.inf, jnp.float32)])
    cent128 = jnp.concatenate([cent, jnp.zeros((112,), jnp.float32)])

    ns = 1
    step = n // ns
    outs = [_run_slice(xf[i * step:(i + 1) * step], flips768, hs, bp128,
                       cent128) for i in range(ns)]
    out = jnp.concatenate(outs, axis=0)
    return out.reshape(x.shape).astype(orig_dtype)


# speedup vs baseline: 1.1929x; 1.0926x over previous
import functools

import jax
import jax.numpy as jnp
import numpy as np
from jax import lax
from jax.experimental import pallas as pl
from jax.experimental.pallas import tpu as pltpu
from jax.experimental.pallas import tpu_sc as plsc

_D_IN = 768
_D = 1024
_L = 16
_CH = 256
_ROWS_PER_BLOCK = 1024

_NW = 32
_SC_CHUNK = 32768


def _hadamard_f32(n: int) -> np.ndarray:
    h = np.array([[1.0]], dtype=np.float32)
    while h.shape[0] < n:
        h = np.block([[h, h], [h, -h]])
    return h


def _fwd_body(x_ref, flips_ref, h_ref, o_ref):
    hs = h_ref[...]
    z = x_ref[...] * flips_ref[...]
    dims = (((1,), (0,)), ((), ()))

    hb = hs.astype(jnp.bfloat16)

    def hmul(za):
        za_hi = za.astype(jnp.bfloat16)
        za_lo = (za - za_hi.astype(jnp.float32)).astype(jnp.bfloat16)
        return (lax.dot_general(za_hi, hb, dims,
                                preferred_element_type=jnp.float32)
                + lax.dot_general(za_lo, hb, dims,
                                  preferred_element_type=jnp.float32))

    m0 = hmul(z[:, 0 * _CH:1 * _CH])
    m1 = hmul(z[:, 1 * _CH:2 * _CH])
    m2 = hmul(z[:, 2 * _CH:3 * _CH])
    b0 = m0 + m1
    b1 = m0 - m1
    o_ref[:, 0 * _CH:1 * _CH] = b0 + m2
    o_ref[:, 1 * _CH:2 * _CH] = b1 + m2
    o_ref[:, 2 * _CH:3 * _CH] = b0 - m2
    o_ref[:, 3 * _CH:4 * _CH] = b1 - m2


def _inv_body(q_ref, flips_ref, h_ref, o_ref):
    hs = h_ref[...]
    q = q_ref[...]
    q0 = q[:, 0 * _CH:1 * _CH]
    q1 = q[:, 1 * _CH:2 * _CH]
    q2 = q[:, 2 * _CH:3 * _CH]
    q3 = q[:, 3 * _CH:4 * _CH]
    u0 = q0 + q1
    u1 = q0 - q1
    u2 = q2 + q3
    u3 = q2 - q3
    dims = (((1,), (0,)), ((), ()))

    def hmul(pa):
        return lax.dot_general(pa, hs, dims,
                               preferred_element_type=jnp.float32)

    f = flips_ref[...]
    o_ref[:, 0 * _CH:1 * _CH] = hmul(u0 + u2) * f[:, 0 * _CH:1 * _CH]
    o_ref[:, 1 * _CH:2 * _CH] = hmul(u1 + u3) * f[:, 1 * _CH:2 * _CH]
    o_ref[:, 2 * _CH:3 * _CH] = hmul(u0 - u2) * f[:, 2 * _CH:3 * _CH]


_SC_ROWS = 32


def _sc_quant_body(y_hbm, bp_hbm, cent_hbm, out_hbm,
                   bpv, centv, b0, b1, b2,
                   si0, si1, si2, so0, so1, so2):
    n_rows = y_hbm.shape[0]
    per_w = n_rows // _NW
    n_chunks = per_w // _SC_ROWS
    wid = lax.axis_index("s") * 2 + lax.axis_index("c")
    base = wid * per_w

    pltpu.sync_copy(bp_hbm, bpv)
    pltpu.sync_copy(cent_hbm, centv)

    bufs = (b0, b1, b2)
    sin = (si0, si1, si2)
    sout = (so0, so1, so2)

    def splat(ref, i):
        return plsc.load_gather(ref, [jnp.full((16,), i, jnp.int32)])

    bp3v = splat(bpv, 3)
    bp7v = splat(bpv, 7)
    bp11v = splat(bpv, 11)
    c8 = jnp.full((16,), 8, jnp.int32)
    c0 = jnp.full((16,), 0, jnp.int32)
    c4 = jnp.full((16,), 4, jnp.int32)
    c2 = jnp.full((16,), 2, jnp.int32)
    c1 = jnp.full((16,), 1, jnp.int32)

    def quantize_chunk(buf):
        @plsc.parallel_loop(0, _SC_ROWS * (_D // 16), 1, unroll=8)
        def _(i):
            row = lax.shift_right_logical(i, 6)
            col = lax.shift_left(jnp.bitwise_and(i, 63), 4)
            yv = buf[row, pl.ds(col, 16)]
            m8 = yv > bp7v
            lo = jnp.where(m8, c8, c0)
            probe = jnp.where(m8, bp11v, bp3v)
            lo = lo + jnp.where(yv > probe, c4, c0)
            lo = lo + jnp.where(yv > plsc.load_gather(bpv, [lo + 1]), c2, c0)
            lo = lo + jnp.where(yv > plsc.load_gather(bpv, [lo]), c1, c0)
            buf[row, pl.ds(col, 16)] = plsc.load_gather(centv, [lo])

    in_h = [None, None, None]
    out_h = [None, None, None]
    in_h[0] = pltpu.async_copy(y_hbm.at[pl.ds(base, _SC_ROWS)], b0, si0)
    in_h[1] = pltpu.async_copy(
        y_hbm.at[pl.ds(base + _SC_ROWS, _SC_ROWS)], b1, si1)
    for c in range(n_chunks):
        k = c % 3
        in_h[k].wait()
        quantize_chunk(bufs[k])
        out_h[k] = pltpu.async_copy(
            bufs[k], out_hbm.at[pl.ds(base + c * _SC_ROWS, _SC_ROWS)],
            sout[k])
        if c + 2 < n_chunks:
            k2 = (c + 2) % 3
            if out_h[k2] is not None:
                out_h[k2].wait()
            in_h[k2] = pltpu.async_copy(
                y_hbm.at[pl.ds(base + (c + 2) * _SC_ROWS, _SC_ROWS)],
                bufs[k2], sin[k2])
    for k in range(3):
        if out_h[k] is not None:
            out_h[k].wait()


def _run_slice(xf, flips768, hs, bp128, cent128):
    n = xf.shape[0]
    r = _ROWS_PER_BLOCK
    grid = (n // r,)

    y = pl.pallas_call(
        _fwd_body,
        grid=grid,
        in_specs=[
            pl.BlockSpec((r, _D_IN), lambda i: (i, 0)),
            pl.BlockSpec((1, _D_IN), lambda i: (0, 0)),
            pl.BlockSpec((_CH, _CH), lambda i: (0, 0)),
        ],
        out_specs=pl.BlockSpec((r, _D), lambda i: (i, 0)),
        out_shape=jax.ShapeDtypeStruct((n, _D), jnp.float32),
        compiler_params=pltpu.CompilerParams(
            dimension_semantics=("arbitrary",)),
    )(xf, flips768, hs)

    mesh = plsc.VectorSubcoreMesh(core_axis_name="c", subcore_axis_name="s")
    sc_quant = functools.partial(
        pl.kernel,
        mesh=mesh,
        compiler_params=pltpu.CompilerParams(
            needs_layout_passes=False, use_tc_tiling_on_sc=True),
        out_type=jax.ShapeDtypeStruct((n, _D), jnp.float32),
        scratch_types=[
            pltpu.VMEM((128,), jnp.float32),
            pltpu.VMEM((128,), jnp.float32),
            pltpu.VMEM((_SC_ROWS, _D), jnp.float32),
            pltpu.VMEM((_SC_ROWS, _D), jnp.float32),
            pltpu.VMEM((_SC_ROWS, _D), jnp.float32),
            pltpu.SemaphoreType.DMA,
            pltpu.SemaphoreType.DMA,
            pltpu.SemaphoreType.DMA,
            pltpu.SemaphoreType.DMA,
            pltpu.SemaphoreType.DMA,
            pltpu.SemaphoreType.DMA,
        ],
    )(_sc_quant_body)
    q = sc_quant(y, bp128, cent128)

    out = pl.pallas_call(
        _inv_body,
        grid=grid,
        in_specs=[
            pl.BlockSpec((r, _D), lambda i: (i, 0)),
            pl.BlockSpec((1, _D_IN), lambda i: (0, 0)),
            pl.BlockSpec((_CH, _CH), lambda i: (0, 0)),
        ],
        out_specs=pl.BlockSpec((r, _D_IN), lambda i: (i, 0)),
        out_shape=jax.ShapeDtypeStruct((n, _D_IN), jnp.float32),
        compiler_params=pltpu.CompilerParams(
            dimension_semantics=("arbitrary",)),
    )(q, flips768, hs)
    return out


@jax.jit
def kernel(x, flips, bp, cent):
    orig_dtype = x.dtype
    n = x.shape[0] * x.shape[1]
    xf = x.reshape(n, _D_IN).astype(jnp.float32)
    hs = jnp.asarray(_hadamard_f32(_CH) * (1.0 / 32.0))
    flips768 = flips[:_D_IN].reshape(1, _D_IN)
    bp128 = jnp.concatenate([bp, jnp.full((113,), jnp.inf, jnp.float32)])
    cent128 = jnp.concatenate([cent, jnp.zeros((112,), jnp.float32)])

    ns = 1
    step = n // ns
    outs = [_run_slice(xf[i * step:(i + 1) * step], flips768, hs, bp128,
                       cent128) for i in range(ns)]
    out = jnp.concatenate(outs, axis=0)
    return out.reshape(x.shape).astype(orig_dtype)


# speedup vs baseline: 1.2106x; 1.0148x over previous
import functools

import jax
import jax.numpy as jnp
import numpy as np
from jax import lax
from jax.experimental import pallas as pl
from jax.experimental.pallas import tpu as pltpu
from jax.experimental.pallas import tpu_sc as plsc

_D_IN = 768
_D = 1024
_L = 16
_CH = 256
_ROWS_PER_BLOCK = 2048

_NW = 32
_SC_CHUNK = 32768


def _hadamard_f32(n: int) -> np.ndarray:
    h = np.array([[1.0]], dtype=np.float32)
    while h.shape[0] < n:
        h = np.block([[h, h], [h, -h]])
    return h


def _fwd_body(x_ref, flips_ref, h_ref, o_ref):
    hs = h_ref[...]
    z = x_ref[...] * flips_ref[...]
    dims = (((1,), (0,)), ((), ()))

    hb = hs.astype(jnp.bfloat16)

    def hmul(za):
        za_hi = za.astype(jnp.bfloat16)
        za_lo = (za - za_hi.astype(jnp.float32)).astype(jnp.bfloat16)
        return (lax.dot_general(za_hi, hb, dims,
                                preferred_element_type=jnp.float32)
                + lax.dot_general(za_lo, hb, dims,
                                  preferred_element_type=jnp.float32))

    m0 = hmul(z[:, 0 * _CH:1 * _CH])
    m1 = hmul(z[:, 1 * _CH:2 * _CH])
    m2 = hmul(z[:, 2 * _CH:3 * _CH])
    b0 = m0 + m1
    b1 = m0 - m1
    o_ref[:, 0 * _CH:1 * _CH] = b0 + m2
    o_ref[:, 1 * _CH:2 * _CH] = b1 + m2
    o_ref[:, 2 * _CH:3 * _CH] = b0 - m2
    o_ref[:, 3 * _CH:4 * _CH] = b1 - m2


def _inv_body(q_ref, flips_ref, h_ref, o_ref):
    hs = h_ref[...]
    q = q_ref[...]
    q0 = q[:, 0 * _CH:1 * _CH]
    q1 = q[:, 1 * _CH:2 * _CH]
    q2 = q[:, 2 * _CH:3 * _CH]
    q3 = q[:, 3 * _CH:4 * _CH]
    u0 = q0 + q1
    u1 = q0 - q1
    u2 = q2 + q3
    u3 = q2 - q3
    dims = (((1,), (0,)), ((), ()))

    def hmul(pa):
        return lax.dot_general(pa, hs, dims,
                               preferred_element_type=jnp.float32)

    f = flips_ref[...]
    o_ref[:, 0 * _CH:1 * _CH] = hmul(u0 + u2) * f[:, 0 * _CH:1 * _CH]
    o_ref[:, 1 * _CH:2 * _CH] = hmul(u1 + u3) * f[:, 1 * _CH:2 * _CH]
    o_ref[:, 2 * _CH:3 * _CH] = hmul(u0 - u2) * f[:, 2 * _CH:3 * _CH]


_SC_ROWS = 32


def _sc_quant_body(y_hbm, bp_hbm, cent_hbm, out_hbm,
                   bpv, centv, b0, b1, b2,
                   si0, si1, si2, so0, so1, so2):
    n_rows = y_hbm.shape[0]
    per_w = n_rows // _NW
    n_chunks = per_w // _SC_ROWS
    wid = lax.axis_index("s") * 2 + lax.axis_index("c")
    base = wid * per_w

    pltpu.sync_copy(bp_hbm, bpv)
    pltpu.sync_copy(cent_hbm, centv)

    bufs = (b0, b1, b2)
    sin = (si0, si1, si2)
    sout = (so0, so1, so2)

    def splat(ref, i):
        return plsc.load_gather(ref, [jnp.full((16,), i, jnp.int32)])

    bp3v = splat(bpv, 3)
    bp7v = splat(bpv, 7)
    bp11v = splat(bpv, 11)
    c8 = jnp.full((16,), 8, jnp.int32)
    c0 = jnp.full((16,), 0, jnp.int32)
    c4 = jnp.full((16,), 4, jnp.int32)
    c2 = jnp.full((16,), 2, jnp.int32)
    c1 = jnp.full((16,), 1, jnp.int32)

    def quantize_chunk(buf):
        @plsc.parallel_loop(0, _SC_ROWS * (_D // 16), 1, unroll=8)
        def _(i):
            row = lax.shift_right_logical(i, 6)
            col = lax.shift_left(jnp.bitwise_and(i, 63), 4)
            yv = buf[row, pl.ds(col, 16)]
            m8 = yv > bp7v
            lo = jnp.where(m8, c8, c0)
            probe = jnp.where(m8, bp11v, bp3v)
            lo = lo + jnp.where(yv > probe, c4, c0)
            lo = lo + jnp.where(yv > plsc.load_gather(bpv, [lo + 1]), c2, c0)
            lo = lo + jnp.where(yv > plsc.load_gather(bpv, [lo]), c1, c0)
            buf[row, pl.ds(col, 16)] = plsc.load_gather(centv, [lo])

    in_h = [None, None, None]
    out_h = [None, None, None]
    in_h[0] = pltpu.async_copy(y_hbm.at[pl.ds(base, _SC_ROWS)], b0, si0)
    in_h[1] = pltpu.async_copy(
        y_hbm.at[pl.ds(base + _SC_ROWS, _SC_ROWS)], b1, si1)
    for c in range(n_chunks):
        k = c % 3
        in_h[k].wait()
        quantize_chunk(bufs[k])
        out_h[k] = pltpu.async_copy(
            bufs[k], out_hbm.at[pl.ds(base + c * _SC_ROWS, _SC_ROWS)],
            sout[k])
        if c + 2 < n_chunks:
            k2 = (c + 2) % 3
            if out_h[k2] is not None:
                out_h[k2].wait()
            in_h[k2] = pltpu.async_copy(
                y_hbm.at[pl.ds(base + (c + 2) * _SC_ROWS, _SC_ROWS)],
                bufs[k2], sin[k2])
    for k in range(3):
        if out_h[k] is not None:
            out_h[k].wait()


def _run_slice(xf, flips768, hs, bp128, cent128):
    n = xf.shape[0]
    r = _ROWS_PER_BLOCK
    grid = (n // r,)

    y = pl.pallas_call(
        _fwd_body,
        grid=grid,
        in_specs=[
            pl.BlockSpec((r, _D_IN), lambda i: (i, 0)),
            pl.BlockSpec((1, _D_IN), lambda i: (0, 0)),
            pl.BlockSpec((_CH, _CH), lambda i: (0, 0)),
        ],
        out_specs=pl.BlockSpec((r, _D), lambda i: (i, 0)),
        out_shape=jax.ShapeDtypeStruct((n, _D), jnp.float32),
        compiler_params=pltpu.CompilerParams(
            dimension_semantics=("arbitrary",)),
    )(xf, flips768, hs)

    mesh = plsc.VectorSubcoreMesh(core_axis_name="c", subcore_axis_name="s")
    sc_quant = functools.partial(
        pl.kernel,
        mesh=mesh,
        compiler_params=pltpu.CompilerParams(
            needs_layout_passes=False, use_tc_tiling_on_sc=True),
        out_type=jax.ShapeDtypeStruct((n, _D), jnp.float32),
        scratch_types=[
            pltpu.VMEM((128,), jnp.float32),
            pltpu.VMEM((128,), jnp.float32),
            pltpu.VMEM((_SC_ROWS, _D), jnp.float32),
            pltpu.VMEM((_SC_ROWS, _D), jnp.float32),
            pltpu.VMEM((_SC_ROWS, _D), jnp.float32),
            pltpu.SemaphoreType.DMA,
            pltpu.SemaphoreType.DMA,
            pltpu.SemaphoreType.DMA,
            pltpu.SemaphoreType.DMA,
            pltpu.SemaphoreType.DMA,
            pltpu.SemaphoreType.DMA,
        ],
    )(_sc_quant_body)
    q = sc_quant(y, bp128, cent128)

    out = pl.pallas_call(
        _inv_body,
        grid=grid,
        in_specs=[
            pl.BlockSpec((r, _D), lambda i: (i, 0)),
            pl.BlockSpec((1, _D_IN), lambda i: (0, 0)),
            pl.BlockSpec((_CH, _CH), lambda i: (0, 0)),
        ],
        out_specs=pl.BlockSpec((r, _D_IN), lambda i: (i, 0)),
        out_shape=jax.ShapeDtypeStruct((n, _D_IN), jnp.float32),
        compiler_params=pltpu.CompilerParams(
            dimension_semantics=("arbitrary",)),
    )(q, flips768, hs)
    return out


@jax.jit
def kernel(x, flips, bp, cent):
    orig_dtype = x.dtype
    n = x.shape[0] * x.shape[1]
    xf = x.reshape(n, _D_IN).astype(jnp.float32)
    hs = jnp.asarray(_hadamard_f32(_CH) * (1.0 / 32.0))
    flips768 = flips[:_D_IN].reshape(1, _D_IN)
    bp128 = jnp.concatenate([bp, jnp.full((113,), jnp.inf, jnp.float32)])
    cent128 = jnp.concatenate([cent, jnp.zeros((112,), jnp.float32)])

    ns = 1
    step = n // ns
    outs = [_run_slice(xf[i * step:(i + 1) * step], flips768, hs, bp128,
                       cent128) for i in range(ns)]
    out = jnp.concatenate(outs, axis=0)
    return out.reshape(x.shape).astype(orig_dtype)
